# baseline (speedup 1.0000x reference)
import jax
import jax.numpy as jnp
from jax import lax
from jax.experimental import pallas as pl
from jax.experimental.pallas import tpu as pltpu
from jax.experimental.pallas import tpu_sc as plsc

N = 10000
E = 320000
D = 128

NC = 2
NS = 16
C = 128
CH = C // 2
NCH = 160
EPAD = NS * NCH * C
NPAD = 10240
NH = NPAD // NC
DUM = NH
ACCR = NH + 8
RPT = NH // NS
DPT = NPAD // NS
RBLK = 1024
L = 16
NB = 2


def _make_sc_agg(first: bool):
    out_type = [jax.ShapeDtypeStruct((NC, NH, D), jnp.float32)]
    if first:
        out_type += [
            jax.ShapeDtypeStruct((NPAD,), jnp.float32),
            jax.ShapeDtypeStruct((NC, NS, NCH, C), jnp.int32),
        ]
    scratch = [
        pltpu.VMEM((NCH, C), jnp.int32),
        pltpu.VMEM((NCH, C), jnp.int32),
        pltpu.VMEM((NB, C, D), jnp.float32),
        pltpu.VMEM_SHARED((ACCR, D), jnp.float32),
        pltpu.SemaphoreType.DMA,
    ]
    if first:
        scratch += [
            pltpu.VMEM((C,), jnp.float32),
            pltpu.VMEM((DPT,), jnp.float32),
            pltpu.VMEM_SHARED((NPAD,), jnp.float32),
        ]

    mesh = plsc.VectorSubcoreMesh(core_axis_name="c", subcore_axis_name="s")

    def body(g_hbm, src_hbm, dst_hbm, *rest):
        if first:
            (s_out, deg_out, dloc_out, src_v, dst_v, rows, acc,
             gsem, ones, zd, dacc) = rest
        else:
            s_out, src_v, dst_v, rows, acc, gsem = rest
        c = lax.axis_index("c")
        s = lax.axis_index("s")

        pltpu.sync_copy(src_hbm.at[s], src_v)
        if first:
            pltpu.sync_copy(dst_hbm.at[s], dst_v)
            base = c * NH

            def adj(j, _):
                for k in range(C // L):
                    v = dst_v[j, pl.ds(k * L, L)] - base
                    ok = (v >= 0) & (v < NH)
                    dst_v[j, pl.ds(k * L, L)] = jnp.where(ok, v, DUM)
                return 0
            lax.fori_loop(0, NCH, adj, 0)
            pltpu.sync_copy(dst_v, dloc_out.at[c, s])
        else:
            pltpu.sync_copy(dst_hbm.at[c, s], dst_v)

        def zrow(i, _):
            for k in range(D // L):
                rows[0, i, pl.ds(k * L, L)] = jnp.zeros((L,), jnp.float32)
            return 0
        lax.fori_loop(0, C, zrow, 0)
        for k in range(RPT // C):
            pltpu.sync_copy(rows.at[0], acc.at[pl.ds(s * RPT + k * C, C)])
        rem = RPT % C
        if rem:
            pltpu.sync_copy(rows.at[0].at[pl.ds(0, rem)],
                            acc.at[pl.ds(s * RPT + (RPT // C) * C, rem)])
        if first:
            @pl.when(c == 0)
            def _():
                def zr(i, _):
                    zd[pl.ds(i * L, L)] = jnp.zeros((L,), jnp.float32)
                    return 0
                lax.fori_loop(0, DPT // L, zr, 0)
                pltpu.sync_copy(zd, dacc.at[pl.ds(s * DPT, DPT)])
            for k in range(C // L):
                ones[pl.ds(k * L, L)] = jnp.ones((L,), jnp.float32)
        plsc.subcore_barrier()

        def gather(j, b):
            for h in range(2):
                pltpu.async_copy(
                    g_hbm.at[src_v.at[j, pl.ds(h * CH, CH)]],
                    rows.at[b, pl.ds(h * CH, CH)], gsem)

        def wait_g(j, b):
            for h in range(2):
                pltpu.make_async_copy(
                    g_hbm.at[src_v.at[j, pl.ds(h * CH, CH)]],
                    rows.at[b, pl.ds(h * CH, CH)], gsem).wait()

        def scatter(j, b):
            pltpu.sync_copy(rows.at[b], acc.at[dst_v.at[j]], add=True)

        if first:
            def deg_scatter(j):
                pltpu.sync_copy(ones, dacc.at[src_v.at[j]], add=True)

        gather(0, 0)

        def outer(o, _):
            for b in range(NB):
                j = NB * o + b
                wait_g(j, b)
                gather(j + 1, 1 - b)
                scatter(j, b)
                if first:
                    @pl.when(c == 0)
                    def _(j=j):
                        deg_scatter(j)
            return 0
        lax.fori_loop(0, NCH // NB - 1, outer, 0)

        for b in range(NB):
            j = NCH - NB + b
            wait_g(j, b)
            if b == 0:
                gather(NCH - 1, (NCH - 1) % NB)
            scatter(j, b)
            if first:
                @pl.when(c == 0)
                def _(j=j):
                    deg_scatter(j)

        plsc.subcore_barrier()
        pltpu.sync_copy(acc.at[pl.ds(s * RPT, RPT)],
                        s_out.at[c, pl.ds(s * RPT, RPT)])
        if first:
            @pl.when(c == 0)
            def _():
                pltpu.sync_copy(dacc.at[pl.ds(s * DPT, DPT)],
                                deg_out.at[pl.ds(s * DPT, DPT)])

    return pl.kernel(body, out_type=out_type, mesh=mesh,
                     scratch_types=scratch, name="sc_agg")


_sc_agg_first = _make_sc_agg(True)
_sc_agg_next = _make_sc_agg(False)


def _mm2_body(x_ref, wa_ref, wb_ref, ga_ref, gb_ref):
    x = x_ref[...]
    dn = (((1,), (1,)), ((), ()))
    ga_ref[...] = lax.dot_general(x, wa_ref[...], dn,
                                  preferred_element_type=jnp.float32)
    gb_ref[...] = lax.dot_general(x, wb_ref[...], dn,
                                  preferred_element_type=jnp.float32)


def _tc_pre(xp, wa, wb):
    blk_r = pl.BlockSpec((RBLK, D), lambda i: (i, 0))
    blk_w = pl.BlockSpec((D, D), lambda i: (0, 0))
    return pl.pallas_call(
        _mm2_body,
        grid=(NPAD // RBLK,),
        in_specs=[blk_r, blk_w, blk_w],
        out_specs=[blk_r, blk_r],
        out_shape=[jax.ShapeDtypeStruct((NPAD, D), jnp.float32)] * 2,
    )(xp, wa, wb)


def _fuse_h(sp_ref, r_ref, deg_ref, b_ref):
    t = sp_ref[...] + r_ref[...] + b_ref[...]
    rows = (jax.lax.broadcasted_iota(jnp.int32, (t.shape[0], 1), 0)
            + pl.program_id(0) * t.shape[0])
    dg = jnp.where(rows < N, deg_ref[...], 0.0)
    return jnp.maximum(t * dg, 0.0)


def _mid_body(sp_ref, r_ref, deg_ref, b_ref, wa_ref, wb_ref, ga_ref, gb_ref):
    h = _fuse_h(sp_ref, r_ref, deg_ref, b_ref)
    dn = (((1,), (1,)), ((), ()))
    ga_ref[...] = lax.dot_general(h, wa_ref[...], dn,
                                  preferred_element_type=jnp.float32)
    gb_ref[...] = lax.dot_general(h, wb_ref[...], dn,
                                  preferred_element_type=jnp.float32)


def _fin_body(sp_ref, r_ref, deg_ref, b_ref, wl_ref, bl_ref, o_ref):
    h = _fuse_h(sp_ref, r_ref, deg_ref, b_ref)
    dn = (((1,), (1,)), ((), ()))
    o_ref[...] = lax.dot_general(h, wl_ref[...], dn,
                                 preferred_element_type=jnp.float32) + bl_ref[...]


def _tc_specs():
    blk_r = pl.BlockSpec((RBLK, D), lambda i: (i, 0))
    blk_dg = pl.BlockSpec((RBLK, 1), lambda i: (i, 0))
    blk_b = pl.BlockSpec((1, D), lambda i: (0, 0))
    blk_w = pl.BlockSpec((D, D), lambda i: (0, 0))
    return blk_r, blk_dg, blk_b, blk_w


def _tc_mid(sp, r, degt, brel, wa, wb):
    blk_r, blk_dg, blk_b, blk_w = _tc_specs()
    return pl.pallas_call(
        _mid_body,
        grid=(NPAD // RBLK,),
        in_specs=[blk_r, blk_r, blk_dg, blk_b, blk_w, blk_w],
        out_specs=[blk_r, blk_r],
        out_shape=[jax.ShapeDtypeStruct((NPAD, D), jnp.float32)] * 2,
    )(sp, r, degt, brel, wa, wb)


def _tc_fin(sp, r, degt, brel, wl, bl):
    blk_r, blk_dg, blk_b, blk_w = _tc_specs()
    return pl.pallas_call(
        _fin_body,
        grid=(NPAD // RBLK,),
        in_specs=[blk_r, blk_r, blk_dg, blk_b, blk_w, blk_b],
        out_specs=blk_r,
        out_shape=jax.ShapeDtypeStruct((NPAD, D), jnp.float32),
    )(sp, r, degt, brel, wl, bl)


def kernel(x, edge_index, Wrel0, brel0, Wroot0, Wrel1, brel1, Wroot1,
           Wrel2, brel2, Wroot2, Wlin, blin):
    xp = jnp.pad(x, ((0, NPAD - N), (0, 0)))
    pad = jnp.full((EPAD - E,), N, dtype=jnp.int32)
    src3 = jnp.concatenate([edge_index[0], pad]).reshape(NS, NCH, C)
    dst3 = jnp.concatenate([edge_index[1], pad]).reshape(NS, NCH, C)
    brel0_2 = brel0.reshape(1, D)
    brel1_2 = brel1.reshape(1, D)
    brel2_2 = brel2.reshape(1, D)
    blin_2 = blin.reshape(1, D)

    g0, r0 = _tc_pre(xp, Wrel0, Wroot0)
    s0h, deg, dloc = _sc_agg_first(g0, src3, dst3)
    s0 = s0h.reshape(NPAD, D)
    degt = deg.reshape(NPAD, 1)
    g1, r1 = _tc_mid(s0, r0, degt, brel0_2, Wrel1, Wroot1)
    (s1h,) = _sc_agg_next(g1, src3, dloc)
    g2, r2 = _tc_mid(s1h.reshape(NPAD, D), r1, degt, brel1_2, Wrel2, Wroot2)
    (s2h,) = _sc_agg_next(g2, src3, dloc)
    outp = _tc_fin(s2h.reshape(NPAD, D), r2, degt, brel2_2, Wlin, blin_2)
    return outp[:N]


# speedup vs baseline: 1.1174x; 1.1174x over previous
import jax
import jax.numpy as jnp
from jax import lax
from jax.experimental import pallas as pl
from jax.experimental.pallas import tpu as pltpu
from jax.experimental.pallas import tpu_sc as plsc

N = 10000
E = 320000
D = 128

NC = 2
NS = 16
C = 128
NCH = 160
EW = NCH * C
EPAD = NS * EW
NPAD = 10240
NH = NPAD // NC
DUM = NH
ACCR = NH + 8
RPT = NH // NS
DPT = NPAD // NS
RBLK = 1024
L = 16
NB = 2
CAP = EW + 2 * C
MAXCH = CAP // C
TRASH = CAP - 1


def _prefix16(m):
    p = jnp.where(m, 1, 0)
    lanes = lax.iota(jnp.int32, L)
    for sh in (1, 2, 4, 8):
        g = p[jnp.maximum(lanes - sh, 0)]
        p = p + jnp.where(lanes >= sh, g, 0)
    return p


def _sc_partition():
    out_type = [
        jax.ShapeDtypeStruct((NC, NS, CAP), jnp.int32),
        jax.ShapeDtypeStruct((NC, NS, CAP), jnp.int32),
        jax.ShapeDtypeStruct((NC, NS, L), jnp.int32),
        jax.ShapeDtypeStruct((NC, NPAD), jnp.float32),
    ]
    scratch = [
        pltpu.VMEM((NCH, C), jnp.int32),
        pltpu.VMEM((NCH, C), jnp.int32),
        pltpu.VMEM((C,), jnp.int32),
        pltpu.VMEM((C,), jnp.int32),
        pltpu.VMEM((L,), jnp.int32),
        pltpu.VMEM((C,), jnp.float32),
        pltpu.VMEM((DPT,), jnp.float32),
        pltpu.VMEM_SHARED((NS * CAP,), jnp.int32),
        pltpu.VMEM_SHARED((NS * CAP,), jnp.int32),
        pltpu.VMEM_SHARED((NPAD,), jnp.float32),
    ]
    mesh = plsc.VectorSubcoreMesh(core_axis_name="c", subcore_axis_name="s")

    def body(src_hbm, dst_hbm, psrc_out, pdst_out, cnt_out, deg_out,
             src_v, dst_v, pos_v, lv_v, cnt_v, ones, zd, ssrc, sdst, dacc):
        c = lax.axis_index("c")
        s = lax.axis_index("s")
        base = c * NH
        reg = s * CAP

        pltpu.sync_copy(src_hbm.at[s], src_v)
        pltpu.sync_copy(dst_hbm.at[s], dst_v)

        def zr(i, _):
            zd[pl.ds(i * L, L)] = jnp.zeros((L,), jnp.float32)
            return 0
        lax.fori_loop(0, DPT // L, zr, 0)
        pltpu.sync_copy(zd, dacc.at[pl.ds(s * DPT, DPT)])
        for k in range(C // L):
            ones[pl.ds(k * L, L)] = jnp.ones((L,), jnp.float32)
        plsc.subcore_barrier()

        def dchunk(ch, _):
            pltpu.sync_copy(ones, dacc.at[src_v.at[c * (NCH // NC) + ch]],
                            add=True)
            return 0
        lax.fori_loop(0, NCH // NC, dchunk, 0)

        def row(r, off):
            for k in range(C // L):
                sl = pl.ds(k * L, L)
                lv = dst_v[r, sl] - base
                m = (lv >= 0) & (lv < NH)
                pre = _prefix16(m)
                pos_v[sl] = jnp.where(m, reg + off + pre - 1, reg + TRASH)
                lv_v[sl] = lv
                off = off + pre[L - 1]
            pltpu.sync_copy(src_v.at[r], ssrc.at[pos_v])
            pltpu.sync_copy(lv_v, sdst.at[pos_v])
            return off

        off = lax.fori_loop(0, NCH, row, 0)

        lanes = lax.iota(jnp.int32, L)
        for k in range(C // L):
            sl = pl.ds(k * L, L)
            pos_v[sl] = reg + off + lanes + k * L
            lv_v[sl] = jnp.full((L,), DUM, jnp.int32)
            src_v[0, sl] = jnp.full((L,), N, jnp.int32)
        pltpu.sync_copy(src_v.at[0], ssrc.at[pos_v])
        pltpu.sync_copy(lv_v, sdst.at[pos_v])

        nch = lax.div(off + C - 1, C)
        cnt_v[pl.ds(0, L)] = jnp.zeros((L,), jnp.int32) + nch
        pltpu.sync_copy(cnt_v, cnt_out.at[c, s])
        pltpu.sync_copy(ssrc.at[pl.ds(reg, CAP)], psrc_out.at[c, s])
        pltpu.sync_copy(sdst.at[pl.ds(reg, CAP)], pdst_out.at[c, s])
        plsc.subcore_barrier()
        pltpu.sync_copy(dacc.at[pl.ds(s * DPT, DPT)],
                        deg_out.at[c, pl.ds(s * DPT, DPT)])

    return pl.kernel(body, out_type=out_type, mesh=mesh,
                     scratch_types=scratch, name="sc_partition")


def _sc_agg():
    out_type = [jax.ShapeDtypeStruct((NC, NH, D), jnp.float32)]
    scratch = [
        pltpu.VMEM((CAP,), jnp.int32),
        pltpu.VMEM((MAXCH, C), jnp.int32),
        pltpu.VMEM((L,), jnp.int32),
        pltpu.VMEM((NB, C, D), jnp.float32),
        pltpu.VMEM_SHARED((ACCR, D), jnp.float32),
        pltpu.SemaphoreType.DMA,
    ]
    mesh = plsc.VectorSubcoreMesh(core_axis_name="c", subcore_axis_name="s")

    def body(g_hbm, psrc_hbm, pdst_hbm, cnt_hbm, s_out,
             nsrc, ndst, cnt_v, rows, acc, gsem):
        c = lax.axis_index("c")
        s = lax.axis_index("s")

        pltpu.sync_copy(psrc_hbm.at[c, s], nsrc)
        pltpu.sync_copy(pdst_hbm.at[c, s], ndst)
        pltpu.sync_copy(cnt_hbm.at[c, s], cnt_v)
        nch = cnt_v[pl.ds(0, L)][0]

        def zrow(i, _):
            for k in range(D // L):
                rows[0, i, pl.ds(k * L, L)] = jnp.zeros((L,), jnp.float32)
            return 0
        lax.fori_loop(0, C, zrow, 0)
        for k in range(RPT // C):
            pltpu.sync_copy(rows.at[0], acc.at[pl.ds(s * RPT + k * C, C)])
        rem = RPT % C
        if rem:
            pltpu.sync_copy(rows.at[0].at[pl.ds(0, rem)],
                            acc.at[pl.ds(s * RPT + (RPT // C) * C, rem)])
        plsc.subcore_barrier()

        def gather(j, b):
            pltpu.async_copy(g_hbm.at[nsrc.at[pl.ds(j * C, C)]],
                             rows.at[b], gsem)

        def wait_g(j, b):
            pltpu.make_async_copy(g_hbm.at[nsrc.at[pl.ds(j * C, C)]],
                                  rows.at[b], gsem).wait()

        def scatter(j, b):
            pltpu.sync_copy(rows.at[b], acc.at[ndst.at[j]], add=True)

        @pl.when(nch > 0)
        def _():
            gather(0, 0)

        def chunk(j, _):
            b = lax.rem(j, 2)
            wait_g(j, b)

            @pl.when(j + 1 < nch)
            def _():
                gather(j + 1, 1 - b)
            scatter(j, b)
            return 0
        lax.fori_loop(0, nch, chunk, 0)

        plsc.subcore_barrier()
        pltpu.sync_copy(acc.at[pl.ds(s * RPT, RPT)],
                        s_out.at[c, pl.ds(s * RPT, RPT)])

    return pl.kernel(body, out_type=out_type, mesh=mesh,
                     scratch_types=scratch, name="sc_agg")


_sc_part = _sc_partition()
_sc_seg = _sc_agg()


def _mm2_body(x_ref, wa_ref, wb_ref, ga_ref, gb_ref):
    x = x_ref[...]
    dn = (((1,), (1,)), ((), ()))
    ga_ref[...] = lax.dot_general(x, wa_ref[...], dn,
                                  preferred_element_type=jnp.float32)
    gb_ref[...] = lax.dot_general(x, wb_ref[...], dn,
                                  preferred_element_type=jnp.float32)


def _tc_pre(xp, wa, wb):
    blk_r = pl.BlockSpec((RBLK, D), lambda i: (i, 0))
    blk_w = pl.BlockSpec((D, D), lambda i: (0, 0))
    return pl.pallas_call(
        _mm2_body,
        grid=(NPAD // RBLK,),
        in_specs=[blk_r, blk_w, blk_w],
        out_specs=[blk_r, blk_r],
        out_shape=[jax.ShapeDtypeStruct((NPAD, D), jnp.float32)] * 2,
    )(xp, wa, wb)


def _fuse_h(sp_ref, r_ref, deg_ref, b_ref):
    t = sp_ref[...] + r_ref[...] + b_ref[...]
    dg = deg_ref[:, 0:1] + deg_ref[:, 1:2]
    rows = (jax.lax.broadcasted_iota(jnp.int32, (t.shape[0], 1), 0)
            + pl.program_id(0) * t.shape[0])
    dg = jnp.where(rows < N, dg, 0.0)
    return jnp.maximum(t * dg, 0.0)


def _mid_body(sp_ref, r_ref, deg_ref, b_ref, wa_ref, wb_ref, ga_ref, gb_ref):
    h = _fuse_h(sp_ref, r_ref, deg_ref, b_ref)
    dn = (((1,), (1,)), ((), ()))
    ga_ref[...] = lax.dot_general(h, wa_ref[...], dn,
                                  preferred_element_type=jnp.float32)
    gb_ref[...] = lax.dot_general(h, wb_ref[...], dn,
                                  preferred_element_type=jnp.float32)


def _fin_body(sp_ref, r_ref, deg_ref, b_ref, wl_ref, bl_ref, o_ref):
    h = _fuse_h(sp_ref, r_ref, deg_ref, b_ref)
    dn = (((1,), (1,)), ((), ()))
    o_ref[...] = lax.dot_general(h, wl_ref[...], dn,
                                 preferred_element_type=jnp.float32) + bl_ref[...]


def _tc_specs():
    blk_r = pl.BlockSpec((RBLK, D), lambda i: (i, 0))
    blk_dg = pl.BlockSpec((RBLK, NC), lambda i: (i, 0))
    blk_b = pl.BlockSpec((1, D), lambda i: (0, 0))
    blk_w = pl.BlockSpec((D, D), lambda i: (0, 0))
    return blk_r, blk_dg, blk_b, blk_w


def _tc_mid(sp, r, degt, brel, wa, wb):
    blk_r, blk_dg, blk_b, blk_w = _tc_specs()
    return pl.pallas_call(
        _mid_body,
        grid=(NPAD // RBLK,),
        in_specs=[blk_r, blk_r, blk_dg, blk_b, blk_w, blk_w],
        out_specs=[blk_r, blk_r],
        out_shape=[jax.ShapeDtypeStruct((NPAD, D), jnp.float32)] * 2,
    )(sp, r, degt, brel, wa, wb)


def _tc_fin(sp, r, degt, brel, wl, bl):
    blk_r, blk_dg, blk_b, blk_w = _tc_specs()
    return pl.pallas_call(
        _fin_body,
        grid=(NPAD // RBLK,),
        in_specs=[blk_r, blk_r, blk_dg, blk_b, blk_w, blk_b],
        out_specs=blk_r,
        out_shape=jax.ShapeDtypeStruct((NPAD, D), jnp.float32),
    )(sp, r, degt, brel, wl, bl)


def kernel(x, edge_index, Wrel0, brel0, Wroot0, Wrel1, brel1, Wroot1,
           Wrel2, brel2, Wroot2, Wlin, blin):
    xp = jnp.pad(x, ((0, NPAD - N), (0, 0)))
    pad = jnp.full((EPAD - E,), N, dtype=jnp.int32)
    src3 = jnp.concatenate([edge_index[0], pad]).reshape(NS, NCH, C)
    dst3 = jnp.concatenate([edge_index[1], pad]).reshape(NS, NCH, C)
    brel0_2 = brel0.reshape(1, D)
    brel1_2 = brel1.reshape(1, D)
    brel2_2 = brel2.reshape(1, D)
    blin_2 = blin.reshape(1, D)

    psrc, pdstf, pcnt, degp = _sc_part(src3, dst3)
    pdst = pdstf.reshape(NC, NS, MAXCH, C)
    degt = degp.T

    g0, r0 = _tc_pre(xp, Wrel0, Wroot0)
    (s0h,) = _sc_seg(g0, psrc, pdst, pcnt)
    g1, r1 = _tc_mid(s0h.reshape(NPAD, D), r0, degt, brel0_2, Wrel1, Wroot1)
    (s1h,) = _sc_seg(g1, psrc, pdst, pcnt)
    g2, r2 = _tc_mid(s1h.reshape(NPAD, D), r1, degt, brel1_2, Wrel2, Wroot2)
    (s2h,) = _sc_seg(g2, psrc, pdst, pcnt)
    outp = _tc_fin(s2h.reshape(NPAD, D), r2, degt, brel2_2, Wlin, blin_2)
    return outp[:N]


# speedup vs baseline: 3.0656x; 2.7435x over previous
import jax
import jax.numpy as jnp
from jax import lax
from jax.experimental import pallas as pl
from jax.experimental.pallas import tpu as pltpu
from jax.experimental.pallas import tpu_sc as plsc

N = 10000
E = 320000
D = 128

NC = 2
NS = 16
C = 128
NCH = 160
EW = NCH * C
EPAD = NS * EW
NPAD = 10240
NH = NPAD // NC
DUM = NH
ACCR = NH + 8
RPT = NH // NS
DPT = NPAD // NS
RBLK = 1024
L = 16
NB = 2
CAP = EW + 2 * C
MAXCH = CAP // C
TRASH = CAP - 1


def _prefix16(m):
    p = jnp.where(m, 1, 0)
    lanes = lax.iota(jnp.int32, L)
    for sh in (1, 2, 4, 8):
        g = p[jnp.maximum(lanes - sh, 0)]
        p = p + jnp.where(lanes >= sh, g, 0)
    return p


def _sc_partition():
    out_type = [
        jax.ShapeDtypeStruct((NC, NS, CAP), jnp.int32),
        jax.ShapeDtypeStruct((NC, NS, CAP), jnp.int32),
        jax.ShapeDtypeStruct((NC, NS, L), jnp.int32),
        jax.ShapeDtypeStruct((NC, NPAD), jnp.float32),
    ]
    scratch = [
        pltpu.VMEM((NCH, C), jnp.int32),
        pltpu.VMEM((NCH, C), jnp.int32),
        pltpu.VMEM((C,), jnp.int32),
        pltpu.VMEM((C,), jnp.int32),
        pltpu.VMEM((L,), jnp.int32),
        pltpu.VMEM((C,), jnp.float32),
        pltpu.VMEM((DPT,), jnp.float32),
        pltpu.VMEM_SHARED((NS * CAP,), jnp.int32),
        pltpu.VMEM_SHARED((NS * CAP,), jnp.int32),
        pltpu.VMEM_SHARED((NPAD,), jnp.float32),
    ]
    mesh = plsc.VectorSubcoreMesh(core_axis_name="c", subcore_axis_name="s")

    def body(src_hbm, dst_hbm, psrc_out, pdst_out, cnt_out, deg_out,
             src_v, dst_v, pos_v, lv_v, cnt_v, ones, zd, ssrc, sdst, dacc):
        c = lax.axis_index("c")
        s = lax.axis_index("s")
        base = c * NH
        reg = s * CAP

        pltpu.sync_copy(src_hbm.at[s], src_v)
        pltpu.sync_copy(dst_hbm.at[s], dst_v)

        def zr(i, _):
            zd[pl.ds(i * L, L)] = jnp.zeros((L,), jnp.float32)
            return 0
        lax.fori_loop(0, DPT // L, zr, 0)
        pltpu.sync_copy(zd, dacc.at[pl.ds(s * DPT, DPT)])
        for k in range(C // L):
            ones[pl.ds(k * L, L)] = jnp.ones((L,), jnp.float32)
        plsc.subcore_barrier()

        def dchunk(ch, _):
            pltpu.sync_copy(ones, dacc.at[src_v.at[c * (NCH // NC) + ch]],
                            add=True)
            return 0
        lax.fori_loop(0, NCH // NC, dchunk, 0)

        def row(r, off):
            for k in range(C // L):
                sl = pl.ds(k * L, L)
                lv = dst_v[r, sl] - base
                m = (lv >= 0) & (lv < NH)
                pre = _prefix16(m)
                pos_v[sl] = jnp.where(m, reg + off + pre - 1, reg + TRASH)
                lv_v[sl] = lv
                off = off + pre[L - 1]
            pltpu.sync_copy(src_v.at[r], ssrc.at[pos_v])
            pltpu.sync_copy(lv_v, sdst.at[pos_v])
            return off

        off = lax.fori_loop(0, NCH, row, 0)

        lanes = lax.iota(jnp.int32, L)
        for k in range(C // L):
            sl = pl.ds(k * L, L)
            pos_v[sl] = reg + off + lanes + k * L
            lv_v[sl] = jnp.full((L,), DUM, jnp.int32)
            src_v[0, sl] = jnp.full((L,), N, jnp.int32)
        pltpu.sync_copy(src_v.at[0], ssrc.at[pos_v])
        pltpu.sync_copy(lv_v, sdst.at[pos_v])

        nch = lax.div(off + C - 1, C)
        cnt_v[pl.ds(0, L)] = jnp.zeros((L,), jnp.int32) + nch
        pltpu.sync_copy(cnt_v, cnt_out.at[c, s])
        pltpu.sync_copy(ssrc.at[pl.ds(reg, CAP)], psrc_out.at[c, s])
        pltpu.sync_copy(sdst.at[pl.ds(reg, CAP)], pdst_out.at[c, s])
        plsc.subcore_barrier()
        pltpu.sync_copy(dacc.at[pl.ds(s * DPT, DPT)],
                        deg_out.at[c, pl.ds(s * DPT, DPT)])

    return pl.kernel(body, out_type=out_type, mesh=mesh,
                     scratch_types=scratch, name="sc_partition")


def _sc_agg():
    out_type = [jax.ShapeDtypeStruct((NC, NH, D), jnp.float32)]
    scratch = [
        pltpu.VMEM((CAP,), jnp.int32),
        pltpu.VMEM((MAXCH, C), jnp.int32),
        pltpu.VMEM((L,), jnp.int32),
        pltpu.VMEM((NB, C, D), jnp.float32),
        pltpu.VMEM_SHARED((ACCR, D), jnp.float32),
        pltpu.SemaphoreType.DMA,
    ]
    mesh = plsc.VectorSubcoreMesh(core_axis_name="c", subcore_axis_name="s")

    def body(g_hbm, psrc_hbm, pdst_hbm, cnt_hbm, s_out,
             nsrc, ndst, cnt_v, rows, acc, gsem):
        c = lax.axis_index("c")
        s = lax.axis_index("s")

        pltpu.sync_copy(psrc_hbm.at[c, s], nsrc)
        pltpu.sync_copy(pdst_hbm.at[c, s], ndst)
        pltpu.sync_copy(cnt_hbm.at[c, s], cnt_v)
        nch = cnt_v[pl.ds(0, L)][0]

        def zrow(i, _):
            for k in range(D // L):
                rows[0, i, pl.ds(k * L, L)] = jnp.zeros((L,), jnp.float32)
            return 0
        lax.fori_loop(0, C, zrow, 0)
        for k in range(RPT // C):
            pltpu.sync_copy(rows.at[0], acc.at[pl.ds(s * RPT + k * C, C)])
        rem = RPT % C
        if rem:
            pltpu.sync_copy(rows.at[0].at[pl.ds(0, rem)],
                            acc.at[pl.ds(s * RPT + (RPT // C) * C, rem)])
        plsc.subcore_barrier()

        def gather(j, b):
            pltpu.async_copy(g_hbm.at[nsrc.at[pl.ds(j * C, C)]],
                             rows.at[b], gsem)

        def wait_g(j, b):
            pltpu.make_async_copy(g_hbm.at[nsrc.at[pl.ds(j * C, C)]],
                                  rows.at[b], gsem).wait()

        def scatter(j, b):
            pltpu.sync_copy(rows.at[b], acc.at[ndst.at[j]], add=True)

        @pl.when(nch > 0)
        def _():
            gather(0, 0)

        def chunk(j, _):
            b = lax.rem(j, 2)
            wait_g(j, b)

            @pl.when(j + 1 < nch)
            def _():
                gather(j + 1, 1 - b)
            scatter(j, b)
            return 0
        lax.fori_loop(0, nch, chunk, 0)

        plsc.subcore_barrier()
        pltpu.sync_copy(acc.at[pl.ds(s * RPT, RPT)],
                        s_out.at[c, pl.ds(s * RPT, RPT)])

    return pl.kernel(body, out_type=out_type, mesh=mesh,
                     scratch_types=scratch, name="sc_agg")


_sc_part = _sc_partition()
_sc_seg = _sc_agg()


def _mm2_body(x_ref, wa_ref, wb_ref, ga_ref, gb_ref):
    x = x_ref[...]
    dn = (((1,), (1,)), ((), ()))
    ga_ref[...] = lax.dot_general(x, wa_ref[...], dn,
                                  preferred_element_type=jnp.float32)
    gb_ref[...] = lax.dot_general(x, wb_ref[...], dn,
                                  preferred_element_type=jnp.float32)


def _tc_pre(xp, wa, wb):
    blk_r = pl.BlockSpec((RBLK, D), lambda i: (i, 0))
    blk_w = pl.BlockSpec((D, D), lambda i: (0, 0))
    return pl.pallas_call(
        _mm2_body,
        grid=(NPAD // RBLK,),
        in_specs=[blk_r, blk_w, blk_w],
        out_specs=[blk_r, blk_r],
        out_shape=[jax.ShapeDtypeStruct((NPAD, D), jnp.float32)] * 2,
    )(xp, wa, wb)


def _fuse_h(sp_ref, r_ref, deg_ref, b_ref):
    t = sp_ref[...] + r_ref[...] + b_ref[...]
    dg = deg_ref[:, 0:1] + deg_ref[:, 1:2]
    rows = (jax.lax.broadcasted_iota(jnp.int32, (t.shape[0], 1), 0)
            + pl.program_id(0) * t.shape[0])
    dg = jnp.where(rows < N, dg, 0.0)
    return jnp.maximum(t * dg, 0.0)


def _mid_body(sp_ref, r_ref, deg_ref, b_ref, wa_ref, wb_ref, ga_ref, gb_ref):
    h = _fuse_h(sp_ref, r_ref, deg_ref, b_ref)
    dn = (((1,), (1,)), ((), ()))
    ga_ref[...] = lax.dot_general(h, wa_ref[...], dn,
                                  preferred_element_type=jnp.float32)
    gb_ref[...] = lax.dot_general(h, wb_ref[...], dn,
                                  preferred_element_type=jnp.float32)


def _fin_body(sp_ref, r_ref, deg_ref, b_ref, wl_ref, bl_ref, o_ref):
    h = _fuse_h(sp_ref, r_ref, deg_ref, b_ref)
    dn = (((1,), (1,)), ((), ()))
    o_ref[...] = lax.dot_general(h, wl_ref[...], dn,
                                 preferred_element_type=jnp.float32) + bl_ref[...]


def _tc_specs():
    blk_r = pl.BlockSpec((RBLK, D), lambda i: (i, 0))
    blk_dg = pl.BlockSpec((RBLK, NC), lambda i: (i, 0))
    blk_b = pl.BlockSpec((1, D), lambda i: (0, 0))
    blk_w = pl.BlockSpec((D, D), lambda i: (0, 0))
    return blk_r, blk_dg, blk_b, blk_w


def _tc_mid(sp, r, degt, brel, wa, wb):
    blk_r, blk_dg, blk_b, blk_w = _tc_specs()
    return pl.pallas_call(
        _mid_body,
        grid=(NPAD // RBLK,),
        in_specs=[blk_r, blk_r, blk_dg, blk_b, blk_w, blk_w],
        out_specs=[blk_r, blk_r],
        out_shape=[jax.ShapeDtypeStruct((NPAD, D), jnp.float32)] * 2,
    )(sp, r, degt, brel, wa, wb)


def _tc_fin(sp, r, degt, brel, wl, bl):
    blk_r, blk_dg, blk_b, blk_w = _tc_specs()
    return pl.pallas_call(
        _fin_body,
        grid=(NPAD // RBLK,),
        in_specs=[blk_r, blk_r, blk_dg, blk_b, blk_w, blk_b],
        out_specs=blk_r,
        out_shape=jax.ShapeDtypeStruct((NPAD, D), jnp.float32),
    )(sp, r, degt, brel, wl, bl)


def kernel(x, edge_index, Wrel0, brel0, Wroot0, Wrel1, brel1, Wroot1,
           Wrel2, brel2, Wroot2, Wlin, blin):
    xp = jnp.pad(x, ((0, NPAD - N), (0, 0)))
    pad = jnp.full((EPAD - E,), N, dtype=jnp.int32)
    padd = jnp.full((EPAD - E,), NPAD, dtype=jnp.int32)
    src3 = jnp.concatenate([edge_index[0], pad]).reshape(NS, NCH, C)
    dst3 = jnp.concatenate([edge_index[1], padd]).reshape(NS, NCH, C)
    brel0_2 = brel0.reshape(1, D)
    brel1_2 = brel1.reshape(1, D)
    brel2_2 = brel2.reshape(1, D)
    blin_2 = blin.reshape(1, D)

    psrc, pdstf, pcnt, degp = _sc_part(src3, dst3)
    pdst = pdstf.reshape(NC, NS, MAXCH, C)
    degt = degp.T

    g0, r0 = _tc_pre(xp, Wrel0, Wroot0)
    (s0h,) = _sc_seg(g0, psrc, pdst, pcnt)
    g1, r1 = _tc_mid(s0h.reshape(NPAD, D), r0, degt, brel0_2, Wrel1, Wroot1)
    (s1h,) = _sc_seg(g1, psrc, pdst, pcnt)
    g2, r2 = _tc_mid(s1h.reshape(NPAD, D), r1, degt, brel1_2, Wrel2, Wroot2)
    (s2h,) = _sc_seg(g2, psrc, pdst, pcnt)
    outp = _tc_fin(s2h.reshape(NPAD, D), r2, degt, brel2_2, Wlin, blin_2)
    return outp[:N]


# speedup vs baseline: 3.0737x; 1.0027x over previous
import jax
import jax.numpy as jnp
from jax import lax
from jax.experimental import pallas as pl
from jax.experimental.pallas import tpu as pltpu
from jax.experimental.pallas import tpu_sc as plsc

N = 10000
E = 320000
D = 128

NC = 2
NS = 16
C = 128
NCH = 160
EW = NCH * C
EPAD = NS * EW
NPAD = 10240
NH = NPAD // NC
DUM = NH
ACCR = NH + 8
RPT = NH // NS
DPT = NPAD // NS
RBLK = 1024
L = 16
NB = 2
CAP = EW + 2 * C
MAXCH = CAP // C
TRASH = CAP - 1


def _prefix16(m):
    p = jnp.where(m, 1, 0)
    lanes = lax.iota(jnp.int32, L)
    for sh in (1, 2, 4, 8):
        g = p[jnp.maximum(lanes - sh, 0)]
        p = p + jnp.where(lanes >= sh, g, 0)
    return p


def _sc_partition():
    out_type = [
        jax.ShapeDtypeStruct((NC, NS, CAP), jnp.int32),
        jax.ShapeDtypeStruct((NC, NS, CAP), jnp.int32),
        jax.ShapeDtypeStruct((NC, NS, L), jnp.int32),
        jax.ShapeDtypeStruct((NC, NPAD), jnp.float32),
    ]
    scratch = [
        pltpu.VMEM((NCH, C), jnp.int32),
        pltpu.VMEM((NCH, C), jnp.int32),
        pltpu.VMEM((C,), jnp.int32),
        pltpu.VMEM((C,), jnp.int32),
        pltpu.VMEM((L,), jnp.int32),
        pltpu.VMEM((C,), jnp.float32),
        pltpu.VMEM((DPT,), jnp.float32),
        pltpu.VMEM_SHARED((NS * CAP,), jnp.int32),
        pltpu.VMEM_SHARED((NS * CAP,), jnp.int32),
        pltpu.VMEM_SHARED((NPAD,), jnp.float32),
    ]
    mesh = plsc.VectorSubcoreMesh(core_axis_name="c", subcore_axis_name="s")

    def body(src_hbm, dst_hbm, psrc_out, pdst_out, cnt_out, deg_out,
             src_v, dst_v, pos_v, lv_v, cnt_v, ones, zd, ssrc, sdst, dacc):
        c = lax.axis_index("c")
        s = lax.axis_index("s")
        base = c * NH
        reg = s * CAP

        pltpu.sync_copy(src_hbm.at[s], src_v)
        pltpu.sync_copy(dst_hbm.at[s], dst_v)

        def zr(i, _):
            zd[pl.ds(i * L, L)] = jnp.zeros((L,), jnp.float32)
            return 0
        lax.fori_loop(0, DPT // L, zr, 0)
        pltpu.sync_copy(zd, dacc.at[pl.ds(s * DPT, DPT)])
        for k in range(C // L):
            ones[pl.ds(k * L, L)] = jnp.ones((L,), jnp.float32)
        plsc.subcore_barrier()

        def dchunk(ch, _):
            pltpu.sync_copy(ones, dacc.at[src_v.at[c * (NCH // NC) + ch]],
                            add=True)
            return 0
        lax.fori_loop(0, NCH // NC, dchunk, 0)

        def row(r, off):
            for k in range(C // L):
                sl = pl.ds(k * L, L)
                lv = dst_v[r, sl] - base
                m = (lv >= 0) & (lv < NH)
                pre = _prefix16(m)
                pos_v[sl] = jnp.where(m, reg + off + pre - 1, reg + TRASH)
                lv_v[sl] = lv
                off = off + pre[L - 1]
            pltpu.sync_copy(src_v.at[r], ssrc.at[pos_v])
            pltpu.sync_copy(lv_v, sdst.at[pos_v])
            return off

        off = lax.fori_loop(0, NCH, row, 0)

        lanes = lax.iota(jnp.int32, L)
        for k in range(C // L):
            sl = pl.ds(k * L, L)
            pos_v[sl] = reg + off + lanes + k * L
            lv_v[sl] = jnp.full((L,), DUM, jnp.int32)
            src_v[0, sl] = jnp.full((L,), N, jnp.int32)
        pltpu.sync_copy(src_v.at[0], ssrc.at[pos_v])
        pltpu.sync_copy(lv_v, sdst.at[pos_v])

        nch = lax.div(off + C - 1, C)
        cnt_v[pl.ds(0, L)] = jnp.zeros((L,), jnp.int32) + nch
        pltpu.sync_copy(cnt_v, cnt_out.at[c, s])
        pltpu.sync_copy(ssrc.at[pl.ds(reg, CAP)], psrc_out.at[c, s])
        pltpu.sync_copy(sdst.at[pl.ds(reg, CAP)], pdst_out.at[c, s])
        plsc.subcore_barrier()
        pltpu.sync_copy(dacc.at[pl.ds(s * DPT, DPT)],
                        deg_out.at[c, pl.ds(s * DPT, DPT)])

    return pl.kernel(body, out_type=out_type, mesh=mesh,
                     scratch_types=scratch, name="sc_partition")


def _sc_agg():
    out_type = [jax.ShapeDtypeStruct((NC, NH, D), jnp.float32)]
    scratch = [
        pltpu.VMEM((CAP,), jnp.int32),
        pltpu.VMEM((MAXCH, C), jnp.int32),
        pltpu.VMEM((L,), jnp.int32),
        pltpu.VMEM((NB, C, D), jnp.float32),
        pltpu.VMEM_SHARED((ACCR, D), jnp.float32),
        pltpu.SemaphoreType.DMA,
    ]
    mesh = plsc.VectorSubcoreMesh(core_axis_name="c", subcore_axis_name="s")

    def body(g_hbm, psrc_hbm, pdst_hbm, cnt_hbm, s_out,
             nsrc, ndst, cnt_v, rows, acc, gsem):
        c = lax.axis_index("c")
        s = lax.axis_index("s")

        pltpu.sync_copy(psrc_hbm.at[c, s], nsrc)
        pltpu.sync_copy(pdst_hbm.at[c, s], ndst)
        pltpu.sync_copy(cnt_hbm.at[c, s], cnt_v)
        nch = cnt_v[pl.ds(0, L)][0]

        def zrow(i, _):
            for k in range(D // L):
                rows[0, i, pl.ds(k * L, L)] = jnp.zeros((L,), jnp.float32)
            return 0
        lax.fori_loop(0, C, zrow, 0)
        for k in range(RPT // C):
            pltpu.sync_copy(rows.at[0], acc.at[pl.ds(s * RPT + k * C, C)])
        rem = RPT % C
        if rem:
            pltpu.sync_copy(rows.at[0].at[pl.ds(0, rem)],
                            acc.at[pl.ds(s * RPT + (RPT // C) * C, rem)])
        plsc.subcore_barrier()

        def gather(j, b):
            for h in range(2):
                pltpu.async_copy(
                    g_hbm.at[nsrc.at[pl.ds(j * C + h * (C // 2), C // 2)]],
                    rows.at[b, pl.ds(h * (C // 2), C // 2)], gsem)

        def wait_g(j, b):
            for h in range(2):
                pltpu.make_async_copy(
                    g_hbm.at[nsrc.at[pl.ds(j * C + h * (C // 2), C // 2)]],
                    rows.at[b, pl.ds(h * (C // 2), C // 2)], gsem).wait()

        def scatter(j, b):
            pltpu.sync_copy(rows.at[b], acc.at[ndst.at[j]], add=True)

        @pl.when(nch > 0)
        def _():
            gather(0, 0)

        def chunk(j, _):
            b = lax.rem(j, 2)
            wait_g(j, b)

            @pl.when(j + 1 < nch)
            def _():
                gather(j + 1, 1 - b)
            scatter(j, b)
            return 0
        lax.fori_loop(0, nch, chunk, 0)

        plsc.subcore_barrier()
        pltpu.sync_copy(acc.at[pl.ds(s * RPT, RPT)],
                        s_out.at[c, pl.ds(s * RPT, RPT)])

    return pl.kernel(body, out_type=out_type, mesh=mesh,
                     scratch_types=scratch, name="sc_agg")


_sc_part = _sc_partition()
_sc_seg = _sc_agg()


def _mm2_body(x_ref, wa_ref, wb_ref, ga_ref, gb_ref):
    x = x_ref[...]
    dn = (((1,), (1,)), ((), ()))
    ga_ref[...] = lax.dot_general(x, wa_ref[...], dn,
                                  preferred_element_type=jnp.float32)
    gb_ref[...] = lax.dot_general(x, wb_ref[...], dn,
                                  preferred_element_type=jnp.float32)


def _tc_pre(xp, wa, wb):
    blk_r = pl.BlockSpec((RBLK, D), lambda i: (i, 0))
    blk_w = pl.BlockSpec((D, D), lambda i: (0, 0))
    return pl.pallas_call(
        _mm2_body,
        grid=(NPAD // RBLK,),
        in_specs=[blk_r, blk_w, blk_w],
        out_specs=[blk_r, blk_r],
        out_shape=[jax.ShapeDtypeStruct((NPAD, D), jnp.float32)] * 2,
    )(xp, wa, wb)


def _fuse_h(sp_ref, r_ref, deg_ref, b_ref):
    t = sp_ref[...] + r_ref[...] + b_ref[...]
    dg = deg_ref[:, 0:1] + deg_ref[:, 1:2]
    rows = (jax.lax.broadcasted_iota(jnp.int32, (t.shape[0], 1), 0)
            + pl.program_id(0) * t.shape[0])
    dg = jnp.where(rows < N, dg, 0.0)
    return jnp.maximum(t * dg, 0.0)


def _mid_body(sp_ref, r_ref, deg_ref, b_ref, wa_ref, wb_ref, ga_ref, gb_ref):
    h = _fuse_h(sp_ref, r_ref, deg_ref, b_ref)
    dn = (((1,), (1,)), ((), ()))
    ga_ref[...] = lax.dot_general(h, wa_ref[...], dn,
                                  preferred_element_type=jnp.float32)
    gb_ref[...] = lax.dot_general(h, wb_ref[...], dn,
                                  preferred_element_type=jnp.float32)


def _fin_body(sp_ref, r_ref, deg_ref, b_ref, wl_ref, bl_ref, o_ref):
    h = _fuse_h(sp_ref, r_ref, deg_ref, b_ref)
    dn = (((1,), (1,)), ((), ()))
    o_ref[...] = lax.dot_general(h, wl_ref[...], dn,
                                 preferred_element_type=jnp.float32) + bl_ref[...]


def _tc_specs():
    blk_r = pl.BlockSpec((RBLK, D), lambda i: (i, 0))
    blk_dg = pl.BlockSpec((RBLK, NC), lambda i: (i, 0))
    blk_b = pl.BlockSpec((1, D), lambda i: (0, 0))
    blk_w = pl.BlockSpec((D, D), lambda i: (0, 0))
    return blk_r, blk_dg, blk_b, blk_w


def _tc_mid(sp, r, degt, brel, wa, wb):
    blk_r, blk_dg, blk_b, blk_w = _tc_specs()
    return pl.pallas_call(
        _mid_body,
        grid=(NPAD // RBLK,),
        in_specs=[blk_r, blk_r, blk_dg, blk_b, blk_w, blk_w],
        out_specs=[blk_r, blk_r],
        out_shape=[jax.ShapeDtypeStruct((NPAD, D), jnp.float32)] * 2,
    )(sp, r, degt, brel, wa, wb)


def _tc_fin(sp, r, degt, brel, wl, bl):
    blk_r, blk_dg, blk_b, blk_w = _tc_specs()
    return pl.pallas_call(
        _fin_body,
        grid=(NPAD // RBLK,),
        in_specs=[blk_r, blk_r, blk_dg, blk_b, blk_w, blk_b],
        out_specs=blk_r,
        out_shape=jax.ShapeDtypeStruct((NPAD, D), jnp.float32),
    )(sp, r, degt, brel, wl, bl)


def kernel(x, edge_index, Wrel0, brel0, Wroot0, Wrel1, brel1, Wroot1,
           Wrel2, brel2, Wroot2, Wlin, blin):
    xp = jnp.pad(x, ((0, NPAD - N), (0, 0)))
    pad = jnp.full((EPAD - E,), N, dtype=jnp.int32)
    padd = jnp.full((EPAD - E,), NPAD, dtype=jnp.int32)
    src3 = jnp.concatenate([edge_index[0], pad]).reshape(NS, NCH, C)
    dst3 = jnp.concatenate([edge_index[1], padd]).reshape(NS, NCH, C)
    brel0_2 = brel0.reshape(1, D)
    brel1_2 = brel1.reshape(1, D)
    brel2_2 = brel2.reshape(1, D)
    blin_2 = blin.reshape(1, D)

    psrc, pdstf, pcnt, degp = _sc_part(src3, dst3)
    pdst = pdstf.reshape(NC, NS, MAXCH, C)
    degt = degp.T

    g0, r0 = _tc_pre(xp, Wrel0, Wroot0)
    (s0h,) = _sc_seg(g0, psrc, pdst, pcnt)
    g1, r1 = _tc_mid(s0h.reshape(NPAD, D), r0, degt, brel0_2, Wrel1, Wroot1)
    (s1h,) = _sc_seg(g1, psrc, pdst, pcnt)
    g2, r2 = _tc_mid(s1h.reshape(NPAD, D), r1, degt, brel1_2, Wrel2, Wroot2)
    (s2h,) = _sc_seg(g2, psrc, pdst, pcnt)
    outp = _tc_fin(s2h.reshape(NPAD, D), r2, degt, brel2_2, Wlin, blin_2)
    return outp[:N]


# speedup vs baseline: 3.0962x; 1.0073x over previous
import jax
import jax.numpy as jnp
from jax import lax
from jax.experimental import pallas as pl
from jax.experimental.pallas import tpu as pltpu
from jax.experimental.pallas import tpu_sc as plsc

N = 10000
E = 320000
D = 128

NC = 2
NS = 16
C = 128
NCH = 160
EW = NCH * C
EPAD = NS * EW
NPAD = 10240
NH = NPAD // NC
DUM = NH
ACCR = NH + 8
RPT = NH // NS
DPT = NPAD // NS
RBLK = 1024
L = 16
NB = 2
CAP = EW + 2 * C
MAXCH = CAP // C
TRASH = CAP - 1


def _prefix16(m):
    p = jnp.where(m, 1, 0)
    lanes = lax.iota(jnp.int32, L)
    for sh in (1, 2, 4, 8):
        g = p[jnp.maximum(lanes - sh, 0)]
        p = p + jnp.where(lanes >= sh, g, 0)
    return p


def _sc_partition():
    out_type = [
        jax.ShapeDtypeStruct((NC, NS, CAP), jnp.int32),
        jax.ShapeDtypeStruct((NC, NS, CAP), jnp.int32),
        jax.ShapeDtypeStruct((NC, NS, L), jnp.int32),
        jax.ShapeDtypeStruct((NC, NPAD), jnp.float32),
    ]
    scratch = [
        pltpu.VMEM((NCH, C), jnp.int32),
        pltpu.VMEM((NCH, C), jnp.int32),
        pltpu.VMEM((2, C), jnp.int32),
        pltpu.VMEM((2, C), jnp.int32),
        pltpu.VMEM((L,), jnp.int32),
        pltpu.VMEM((C,), jnp.float32),
        pltpu.VMEM((DPT,), jnp.float32),
        pltpu.VMEM_SHARED((NS * CAP,), jnp.int32),
        pltpu.VMEM_SHARED((NS * CAP,), jnp.int32),
        pltpu.VMEM_SHARED((NPAD,), jnp.float32),
        pltpu.SemaphoreType.DMA,
        pltpu.SemaphoreType.DMA,
    ]
    mesh = plsc.VectorSubcoreMesh(core_axis_name="c", subcore_axis_name="s")

    def body(src_hbm, dst_hbm, psrc_out, pdst_out, cnt_out, deg_out,
             src_v, dst_v, pos_v, lv_v, cnt_v, ones, zd, ssrc, sdst, dacc,
             csem, dsem):
        c = lax.axis_index("c")
        s = lax.axis_index("s")
        base = c * NH
        reg = s * CAP

        pltpu.sync_copy(src_hbm.at[s], src_v)
        pltpu.sync_copy(dst_hbm.at[s], dst_v)

        def zr(i, _):
            zd[pl.ds(i * L, L)] = jnp.zeros((L,), jnp.float32)
            return 0
        lax.fori_loop(0, DPT // L, zr, 0)
        pltpu.sync_copy(zd, dacc.at[pl.ds(s * DPT, DPT)])
        for k in range(C // L):
            ones[pl.ds(k * L, L)] = jnp.ones((L,), jnp.float32)
        plsc.subcore_barrier()

        def deg_fire(ch):
            pltpu.async_copy(ones, dacc.at[src_v.at[c * (NCH // NC) + ch]],
                             dsem, add=True)

        def deg_wait(ch):
            pltpu.make_async_copy(ones, dacc.at[src_v.at[c * (NCH // NC) + ch]],
                                  dsem).wait()

        def dchunk(ch, _):
            deg_fire(ch)

            @pl.when(ch >= 4)
            def _():
                deg_wait(ch - 4)
            return 0
        lax.fori_loop(0, NCH // NC, dchunk, 0)

        def cs_fire(r, br):
            pltpu.async_copy(src_v.at[r], ssrc.at[pos_v.at[br]], csem)
            pltpu.async_copy(lv_v.at[br], sdst.at[pos_v.at[br]], csem)

        def cs_wait(r, br):
            pltpu.make_async_copy(src_v.at[r], ssrc.at[pos_v.at[br]],
                                  csem).wait()
            pltpu.make_async_copy(lv_v.at[br], sdst.at[pos_v.at[br]],
                                  csem).wait()

        def row(r, off):
            br = lax.rem(r, 2)

            @pl.when(r >= 2)
            def _():
                cs_wait(r - 2, br)
            for k in range(C // L):
                sl = pl.ds(k * L, L)
                lv = dst_v[r, sl] - base
                m = (lv >= 0) & (lv < NH)
                pre = _prefix16(m)
                pos_v[br, sl] = jnp.where(m, reg + off + pre - 1, reg + TRASH)
                lv_v[br, sl] = lv
                off = off + pre[L - 1]
            cs_fire(r, br)
            return off

        off = lax.fori_loop(0, NCH, row, 0)
        for r in (NCH - 2, NCH - 1):
            cs_wait(r, r % 2)
        for ch in range(NCH // NC - 4, NCH // NC):
            deg_wait(ch)

        lanes = lax.iota(jnp.int32, L)
        for k in range(C // L):
            sl = pl.ds(k * L, L)
            pos_v[0, sl] = reg + off + lanes + k * L
            lv_v[0, sl] = jnp.full((L,), DUM, jnp.int32)
            src_v[0, sl] = jnp.full((L,), N, jnp.int32)
        pltpu.sync_copy(src_v.at[0], ssrc.at[pos_v.at[0]])
        pltpu.sync_copy(lv_v.at[0], sdst.at[pos_v.at[0]])

        nch = lax.div(off + C - 1, C)
        cnt_v[pl.ds(0, L)] = jnp.zeros((L,), jnp.int32) + nch
        pltpu.sync_copy(cnt_v, cnt_out.at[c, s])
        pltpu.sync_copy(ssrc.at[pl.ds(reg, CAP)], psrc_out.at[c, s])
        pltpu.sync_copy(sdst.at[pl.ds(reg, CAP)], pdst_out.at[c, s])
        plsc.subcore_barrier()
        pltpu.sync_copy(dacc.at[pl.ds(s * DPT, DPT)],
                        deg_out.at[c, pl.ds(s * DPT, DPT)])

    return pl.kernel(body, out_type=out_type, mesh=mesh,
                     scratch_types=scratch, name="sc_partition")


def _sc_agg():
    out_type = [jax.ShapeDtypeStruct((NC, NH, D), jnp.float32)]
    scratch = [
        pltpu.VMEM((CAP,), jnp.int32),
        pltpu.VMEM((MAXCH, C), jnp.int32),
        pltpu.VMEM((L,), jnp.int32),
        pltpu.VMEM((NB, C, D), jnp.float32),
        pltpu.VMEM_SHARED((ACCR, D), jnp.float32),
        pltpu.SemaphoreType.DMA,
    ]
    mesh = plsc.VectorSubcoreMesh(core_axis_name="c", subcore_axis_name="s")

    def body(g_hbm, psrc_hbm, pdst_hbm, cnt_hbm, s_out,
             nsrc, ndst, cnt_v, rows, acc, gsem):
        c = lax.axis_index("c")
        s = lax.axis_index("s")

        pltpu.sync_copy(psrc_hbm.at[c, s], nsrc)
        pltpu.sync_copy(pdst_hbm.at[c, s], ndst)
        pltpu.sync_copy(cnt_hbm.at[c, s], cnt_v)
        nch = cnt_v[pl.ds(0, L)][0]

        def zrow(i, _):
            for k in range(D // L):
                rows[0, i, pl.ds(k * L, L)] = jnp.zeros((L,), jnp.float32)
            return 0
        lax.fori_loop(0, C, zrow, 0)
        for k in range(RPT // C):
            pltpu.sync_copy(rows.at[0], acc.at[pl.ds(s * RPT + k * C, C)])
        rem = RPT % C
        if rem:
            pltpu.sync_copy(rows.at[0].at[pl.ds(0, rem)],
                            acc.at[pl.ds(s * RPT + (RPT // C) * C, rem)])
        plsc.subcore_barrier()

        def gather(j, b):
            for h in range(2):
                pltpu.async_copy(
                    g_hbm.at[nsrc.at[pl.ds(j * C + h * (C // 2), C // 2)]],
                    rows.at[b, pl.ds(h * (C // 2), C // 2)], gsem)

        def wait_g(j, b):
            for h in range(2):
                pltpu.make_async_copy(
                    g_hbm.at[nsrc.at[pl.ds(j * C + h * (C // 2), C // 2)]],
                    rows.at[b, pl.ds(h * (C // 2), C // 2)], gsem).wait()

        def scatter(j, b):
            pltpu.sync_copy(rows.at[b], acc.at[ndst.at[j]], add=True)

        @pl.when(nch > 0)
        def _():
            gather(0, 0)

        def chunk(j, _):
            b = lax.rem(j, 2)
            wait_g(j, b)

            @pl.when(j + 1 < nch)
            def _():
                gather(j + 1, 1 - b)
            scatter(j, b)
            return 0
        lax.fori_loop(0, nch, chunk, 0)

        plsc.subcore_barrier()
        pltpu.sync_copy(acc.at[pl.ds(s * RPT, RPT)],
                        s_out.at[c, pl.ds(s * RPT, RPT)])

    return pl.kernel(body, out_type=out_type, mesh=mesh,
                     scratch_types=scratch, name="sc_agg")


_sc_part = _sc_partition()
_sc_seg = _sc_agg()


def _mm2_body(x_ref, wa_ref, wb_ref, ga_ref, gb_ref):
    x = x_ref[...]
    dn = (((1,), (1,)), ((), ()))
    ga_ref[...] = lax.dot_general(x, wa_ref[...], dn,
                                  preferred_element_type=jnp.float32)
    gb_ref[...] = lax.dot_general(x, wb_ref[...], dn,
                                  preferred_element_type=jnp.float32)


def _tc_pre(xp, wa, wb):
    blk_r = pl.BlockSpec((RBLK, D), lambda i: (i, 0))
    blk_w = pl.BlockSpec((D, D), lambda i: (0, 0))
    return pl.pallas_call(
        _mm2_body,
        grid=(NPAD // RBLK,),
        in_specs=[blk_r, blk_w, blk_w],
        out_specs=[blk_r, blk_r],
        out_shape=[jax.ShapeDtypeStruct((NPAD, D), jnp.float32)] * 2,
    )(xp, wa, wb)


def _fuse_h(sp_ref, r_ref, deg_ref, b_ref):
    t = sp_ref[...] + r_ref[...] + b_ref[...]
    dg = deg_ref[:, 0:1] + deg_ref[:, 1:2]
    rows = (jax.lax.broadcasted_iota(jnp.int32, (t.shape[0], 1), 0)
            + pl.program_id(0) * t.shape[0])
    dg = jnp.where(rows < N, dg, 0.0)
    return jnp.maximum(t * dg, 0.0)


def _mid_body(sp_ref, r_ref, deg_ref, b_ref, wa_ref, wb_ref, ga_ref, gb_ref):
    h = _fuse_h(sp_ref, r_ref, deg_ref, b_ref)
    dn = (((1,), (1,)), ((), ()))
    ga_ref[...] = lax.dot_general(h, wa_ref[...], dn,
                                  preferred_element_type=jnp.float32)
    gb_ref[...] = lax.dot_general(h, wb_ref[...], dn,
                                  preferred_element_type=jnp.float32)


def _fin_body(sp_ref, r_ref, deg_ref, b_ref, wl_ref, bl_ref, o_ref):
    h = _fuse_h(sp_ref, r_ref, deg_ref, b_ref)
    dn = (((1,), (1,)), ((), ()))
    o_ref[...] = lax.dot_general(h, wl_ref[...], dn,
                                 preferred_element_type=jnp.float32) + bl_ref[...]


def _tc_specs():
    blk_r = pl.BlockSpec((RBLK, D), lambda i: (i, 0))
    blk_dg = pl.BlockSpec((RBLK, NC), lambda i: (i, 0))
    blk_b = pl.BlockSpec((1, D), lambda i: (0, 0))
    blk_w = pl.BlockSpec((D, D), lambda i: (0, 0))
    return blk_r, blk_dg, blk_b, blk_w


def _tc_mid(sp, r, degt, brel, wa, wb):
    blk_r, blk_dg, blk_b, blk_w = _tc_specs()
    return pl.pallas_call(
        _mid_body,
        grid=(NPAD // RBLK,),
        in_specs=[blk_r, blk_r, blk_dg, blk_b, blk_w, blk_w],
        out_specs=[blk_r, blk_r],
        out_shape=[jax.ShapeDtypeStruct((NPAD, D), jnp.float32)] * 2,
    )(sp, r, degt, brel, wa, wb)


def _tc_fin(sp, r, degt, brel, wl, bl):
    blk_r, blk_dg, blk_b, blk_w = _tc_specs()
    return pl.pallas_call(
        _fin_body,
        grid=(NPAD // RBLK,),
        in_specs=[blk_r, blk_r, blk_dg, blk_b, blk_w, blk_b],
        out_specs=blk_r,
        out_shape=jax.ShapeDtypeStruct((NPAD, D), jnp.float32),
    )(sp, r, degt, brel, wl, bl)


def kernel(x, edge_index, Wrel0, brel0, Wroot0, Wrel1, brel1, Wroot1,
           Wrel2, brel2, Wroot2, Wlin, blin):
    xp = jnp.pad(x, ((0, NPAD - N), (0, 0)))
    pad = jnp.full((EPAD - E,), N, dtype=jnp.int32)
    padd = jnp.full((EPAD - E,), NPAD, dtype=jnp.int32)
    src3 = jnp.concatenate([edge_index[0], pad]).reshape(NS, NCH, C)
    dst3 = jnp.concatenate([edge_index[1], padd]).reshape(NS, NCH, C)
    brel0_2 = brel0.reshape(1, D)
    brel1_2 = brel1.reshape(1, D)
    brel2_2 = brel2.reshape(1, D)
    blin_2 = blin.reshape(1, D)

    psrc, pdstf, pcnt, degp = _sc_part(src3, dst3)
    pdst = pdstf.reshape(NC, NS, MAXCH, C)
    degt = degp.T

    g0, r0 = _tc_pre(xp, Wrel0, Wroot0)
    (s0h,) = _sc_seg(g0, psrc, pdst, pcnt)
    g1, r1 = _tc_mid(s0h.reshape(NPAD, D), r0, degt, brel0_2, Wrel1, Wroot1)
    (s1h,) = _sc_seg(g1, psrc, pdst, pcnt)
    g2, r2 = _tc_mid(s1h.reshape(NPAD, D), r1, degt, brel1_2, Wrel2, Wroot2)
    (s2h,) = _sc_seg(g2, psrc, pdst, pcnt)
    outp = _tc_fin(s2h.reshape(NPAD, D), r2, degt, brel2_2, Wlin, blin_2)
    return outp[:N]


# speedup vs baseline: 3.1001x; 1.0013x over previous
import jax
import jax.numpy as jnp
from jax import lax
from jax.experimental import pallas as pl
from jax.experimental.pallas import tpu as pltpu
from jax.experimental.pallas import tpu_sc as plsc

N = 10000
E = 320000
D = 128

NC = 2
NS = 16
C = 128
NCH = 160
EW = NCH * C
EPAD = NS * EW
NPAD = 10240
NH = NPAD // NC
DUM = NH
ACCR = NH + 8
RPT = NH // NS
DPT = NPAD // NS
RBLK = 2048
L = 16
NB = 2
CAP = EW + 2 * C
MAXCH = CAP // C
TRASH = CAP - 1


def _prefix16(m):
    p = jnp.where(m, 1, 0)
    lanes = lax.iota(jnp.int32, L)
    for sh in (1, 2, 4, 8):
        g = p[jnp.maximum(lanes - sh, 0)]
        p = p + jnp.where(lanes >= sh, g, 0)
    return p


def _sc_partition():
    out_type = [
        jax.ShapeDtypeStruct((NC, NS, CAP), jnp.int32),
        jax.ShapeDtypeStruct((NC, NS, CAP), jnp.int32),
        jax.ShapeDtypeStruct((NC, NS, L), jnp.int32),
        jax.ShapeDtypeStruct((NC, NPAD), jnp.float32),
    ]
    scratch = [
        pltpu.VMEM((NCH, C), jnp.int32),
        pltpu.VMEM((NCH, C), jnp.int32),
        pltpu.VMEM((2, C), jnp.int32),
        pltpu.VMEM((2, C), jnp.int32),
        pltpu.VMEM((L,), jnp.int32),
        pltpu.VMEM((C,), jnp.float32),
        pltpu.VMEM((DPT,), jnp.float32),
        pltpu.VMEM_SHARED((NS * CAP,), jnp.int32),
        pltpu.VMEM_SHARED((NS * CAP,), jnp.int32),
        pltpu.VMEM_SHARED((NPAD,), jnp.float32),
        pltpu.SemaphoreType.DMA,
        pltpu.SemaphoreType.DMA,
    ]
    mesh = plsc.VectorSubcoreMesh(core_axis_name="c", subcore_axis_name="s")

    def body(src_hbm, dst_hbm, psrc_out, pdst_out, cnt_out, deg_out,
             src_v, dst_v, pos_v, lv_v, cnt_v, ones, zd, ssrc, sdst, dacc,
             csem, dsem):
        c = lax.axis_index("c")
        s = lax.axis_index("s")
        base = c * NH
        reg = s * CAP

        pltpu.sync_copy(src_hbm.at[s], src_v)
        pltpu.sync_copy(dst_hbm.at[s], dst_v)

        def zr(i, _):
            zd[pl.ds(i * L, L)] = jnp.zeros((L,), jnp.float32)
            return 0
        lax.fori_loop(0, DPT // L, zr, 0)
        pltpu.sync_copy(zd, dacc.at[pl.ds(s * DPT, DPT)])
        for k in range(C // L):
            ones[pl.ds(k * L, L)] = jnp.ones((L,), jnp.float32)
        plsc.subcore_barrier()

        def deg_fire(ch):
            pltpu.async_copy(ones, dacc.at[src_v.at[c * (NCH // NC) + ch]],
                             dsem, add=True)

        def deg_wait(ch):
            pltpu.make_async_copy(ones, dacc.at[src_v.at[c * (NCH // NC) + ch]],
                                  dsem).wait()

        def dchunk(ch, _):
            deg_fire(ch)

            @pl.when(ch >= 4)
            def _():
                deg_wait(ch - 4)
            return 0
        lax.fori_loop(0, NCH // NC, dchunk, 0)

        def cs_fire(r, br):
            pltpu.async_copy(src_v.at[r], ssrc.at[pos_v.at[br]], csem)
            pltpu.async_copy(lv_v.at[br], sdst.at[pos_v.at[br]], csem)

        def cs_wait(r, br):
            pltpu.make_async_copy(src_v.at[r], ssrc.at[pos_v.at[br]],
                                  csem).wait()
            pltpu.make_async_copy(lv_v.at[br], sdst.at[pos_v.at[br]],
                                  csem).wait()

        def row(r, off):
            br = lax.rem(r, 2)

            @pl.when(r >= 2)
            def _():
                cs_wait(r - 2, br)
            for k in range(C // L):
                sl = pl.ds(k * L, L)
                lv = dst_v[r, sl] - base
                m = (lv >= 0) & (lv < NH)
                pre = _prefix16(m)
                pos_v[br, sl] = jnp.where(m, reg + off + pre - 1, reg + TRASH)
                lv_v[br, sl] = lv
                off = off + pre[L - 1]
            cs_fire(r, br)
            return off

        off = lax.fori_loop(0, NCH, row, 0)
        for r in (NCH - 2, NCH - 1):
            cs_wait(r, r % 2)
        for ch in range(NCH // NC - 4, NCH // NC):
            deg_wait(ch)

        lanes = lax.iota(jnp.int32, L)
        for k in range(C // L):
            sl = pl.ds(k * L, L)
            pos_v[0, sl] = reg + off + lanes + k * L
            lv_v[0, sl] = jnp.full((L,), DUM, jnp.int32)
            src_v[0, sl] = jnp.full((L,), N, jnp.int32)
        pltpu.sync_copy(src_v.at[0], ssrc.at[pos_v.at[0]])
        pltpu.sync_copy(lv_v.at[0], sdst.at[pos_v.at[0]])

        nch = lax.div(off + C - 1, C)
        cnt_v[pl.ds(0, L)] = jnp.zeros((L,), jnp.int32) + nch
        pltpu.sync_copy(cnt_v, cnt_out.at[c, s])
        pltpu.sync_copy(ssrc.at[pl.ds(reg, CAP)], psrc_out.at[c, s])
        pltpu.sync_copy(sdst.at[pl.ds(reg, CAP)], pdst_out.at[c, s])
        plsc.subcore_barrier()
        pltpu.sync_copy(dacc.at[pl.ds(s * DPT, DPT)],
                        deg_out.at[c, pl.ds(s * DPT, DPT)])

    return pl.kernel(body, out_type=out_type, mesh=mesh,
                     scratch_types=scratch, name="sc_partition")


def _sc_agg():
    out_type = [jax.ShapeDtypeStruct((NC, NH, D), jnp.float32)]
    scratch = [
        pltpu.VMEM((CAP,), jnp.int32),
        pltpu.VMEM((MAXCH, C), jnp.int32),
        pltpu.VMEM((L,), jnp.int32),
        pltpu.VMEM((NB, C, D), jnp.float32),
        pltpu.VMEM_SHARED((ACCR, D), jnp.float32),
        pltpu.SemaphoreType.DMA,
    ]
    mesh = plsc.VectorSubcoreMesh(core_axis_name="c", subcore_axis_name="s")

    def body(g_hbm, psrc_hbm, pdst_hbm, cnt_hbm, s_out,
             nsrc, ndst, cnt_v, rows, acc, gsem):
        c = lax.axis_index("c")
        s = lax.axis_index("s")

        pltpu.sync_copy(psrc_hbm.at[c, s], nsrc)
        pltpu.sync_copy(pdst_hbm.at[c, s], ndst)
        pltpu.sync_copy(cnt_hbm.at[c, s], cnt_v)
        nch = cnt_v[pl.ds(0, L)][0]

        def zrow(i, _):
            for k in range(D // L):
                rows[0, i, pl.ds(k * L, L)] = jnp.zeros((L,), jnp.float32)
            return 0
        lax.fori_loop(0, C, zrow, 0)
        for k in range(RPT // C):
            pltpu.sync_copy(rows.at[0], acc.at[pl.ds(s * RPT + k * C, C)])
        rem = RPT % C
        if rem:
            pltpu.sync_copy(rows.at[0].at[pl.ds(0, rem)],
                            acc.at[pl.ds(s * RPT + (RPT // C) * C, rem)])
        plsc.subcore_barrier()

        def gather(j, b):
            for h in range(2):
                pltpu.async_copy(
                    g_hbm.at[nsrc.at[pl.ds(j * C + h * (C // 2), C // 2)]],
                    rows.at[b, pl.ds(h * (C // 2), C // 2)], gsem)

        def wait_g(j, b):
            for h in range(2):
                pltpu.make_async_copy(
                    g_hbm.at[nsrc.at[pl.ds(j * C + h * (C // 2), C // 2)]],
                    rows.at[b, pl.ds(h * (C // 2), C // 2)], gsem).wait()

        def scatter(j, b):
            pltpu.sync_copy(rows.at[b], acc.at[ndst.at[j]], add=True)

        @pl.when(nch > 0)
        def _():
            gather(0, 0)

        def chunk(j, _):
            b = lax.rem(j, 2)
            wait_g(j, b)

            @pl.when(j + 1 < nch)
            def _():
                gather(j + 1, 1 - b)
            scatter(j, b)
            return 0
        lax.fori_loop(0, nch, chunk, 0)

        plsc.subcore_barrier()
        pltpu.sync_copy(acc.at[pl.ds(s * RPT, RPT)],
                        s_out.at[c, pl.ds(s * RPT, RPT)])

    return pl.kernel(body, out_type=out_type, mesh=mesh,
                     scratch_types=scratch, name="sc_agg")


_sc_part = _sc_partition()
_sc_seg = _sc_agg()


def _mm2_body(x_ref, wa_ref, wb_ref, ga_ref, gb_ref):
    x = x_ref[...]
    dn = (((1,), (1,)), ((), ()))
    ga_ref[...] = lax.dot_general(x, wa_ref[...], dn,
                                  preferred_element_type=jnp.float32)
    gb_ref[...] = lax.dot_general(x, wb_ref[...], dn,
                                  preferred_element_type=jnp.float32)


def _tc_pre(xp, wa, wb):
    blk_r = pl.BlockSpec((RBLK, D), lambda i: (i, 0))
    blk_w = pl.BlockSpec((D, D), lambda i: (0, 0))
    return pl.pallas_call(
        _mm2_body,
        grid=(NPAD // RBLK,),
        in_specs=[blk_r, blk_w, blk_w],
        out_specs=[blk_r, blk_r],
        out_shape=[jax.ShapeDtypeStruct((NPAD, D), jnp.float32)] * 2,
    )(xp, wa, wb)


def _fuse_h(sp_ref, r_ref, deg_ref, b_ref):
    t = sp_ref[...] + r_ref[...] + b_ref[...]
    dg = deg_ref[:, 0:1] + deg_ref[:, 1:2]
    rows = (jax.lax.broadcasted_iota(jnp.int32, (t.shape[0], 1), 0)
            + pl.program_id(0) * t.shape[0])
    dg = jnp.where(rows < N, dg, 0.0)
    return jnp.maximum(t * dg, 0.0)


def _mid_body(sp_ref, r_ref, deg_ref, b_ref, wa_ref, wb_ref, ga_ref, gb_ref):
    h = _fuse_h(sp_ref, r_ref, deg_ref, b_ref)
    dn = (((1,), (1,)), ((), ()))
    ga_ref[...] = lax.dot_general(h, wa_ref[...], dn,
                                  preferred_element_type=jnp.float32)
    gb_ref[...] = lax.dot_general(h, wb_ref[...], dn,
                                  preferred_element_type=jnp.float32)


def _fin_body(sp_ref, r_ref, deg_ref, b_ref, wl_ref, bl_ref, o_ref):
    h = _fuse_h(sp_ref, r_ref, deg_ref, b_ref)
    dn = (((1,), (1,)), ((), ()))
    o_ref[...] = lax.dot_general(h, wl_ref[...], dn,
                                 preferred_element_type=jnp.float32) + bl_ref[...]


def _tc_specs():
    blk_r = pl.BlockSpec((RBLK, D), lambda i: (i, 0))
    blk_dg = pl.BlockSpec((RBLK, NC), lambda i: (i, 0))
    blk_b = pl.BlockSpec((1, D), lambda i: (0, 0))
    blk_w = pl.BlockSpec((D, D), lambda i: (0, 0))
    return blk_r, blk_dg, blk_b, blk_w


def _tc_mid(sp, r, degt, brel, wa, wb):
    blk_r, blk_dg, blk_b, blk_w = _tc_specs()
    return pl.pallas_call(
        _mid_body,
        grid=(NPAD // RBLK,),
        in_specs=[blk_r, blk_r, blk_dg, blk_b, blk_w, blk_w],
        out_specs=[blk_r, blk_r],
        out_shape=[jax.ShapeDtypeStruct((NPAD, D), jnp.float32)] * 2,
    )(sp, r, degt, brel, wa, wb)


def _tc_fin(sp, r, degt, brel, wl, bl):
    blk_r, blk_dg, blk_b, blk_w = _tc_specs()
    return pl.pallas_call(
        _fin_body,
        grid=(NPAD // RBLK,),
        in_specs=[blk_r, blk_r, blk_dg, blk_b, blk_w, blk_b],
        out_specs=blk_r,
        out_shape=jax.ShapeDtypeStruct((NPAD, D), jnp.float32),
    )(sp, r, degt, brel, wl, bl)


def kernel(x, edge_index, Wrel0, brel0, Wroot0, Wrel1, brel1, Wroot1,
           Wrel2, brel2, Wroot2, Wlin, blin):
    xp = jnp.pad(x, ((0, NPAD - N), (0, 0)))
    pad = jnp.full((EPAD - E,), N, dtype=jnp.int32)
    padd = jnp.full((EPAD - E,), NPAD, dtype=jnp.int32)
    src3 = jnp.concatenate([edge_index[0], pad]).reshape(NS, NCH, C)
    dst3 = jnp.concatenate([edge_index[1], padd]).reshape(NS, NCH, C)
    brel0_2 = brel0.reshape(1, D)
    brel1_2 = brel1.reshape(1, D)
    brel2_2 = brel2.reshape(1, D)
    blin_2 = blin.reshape(1, D)

    psrc, pdstf, pcnt, degp = _sc_part(src3, dst3)
    pdst = pdstf.reshape(NC, NS, MAXCH, C)
    degt = degp.T

    g0, r0 = _tc_pre(xp, Wrel0, Wroot0)
    (s0h,) = _sc_seg(g0, psrc, pdst, pcnt)
    g1, r1 = _tc_mid(s0h.reshape(NPAD, D), r0, degt, brel0_2, Wrel1, Wroot1)
    (s1h,) = _sc_seg(g1, psrc, pdst, pcnt)
    g2, r2 = _tc_mid(s1h.reshape(NPAD, D), r1, degt, brel1_2, Wrel2, Wroot2)
    (s2h,) = _sc_seg(g2, psrc, pdst, pcnt)
    outp = _tc_fin(s2h.reshape(NPAD, D), r2, degt, brel2_2, Wlin, blin_2)
    return outp[:N]
